# baseline (speedup 1.0000x reference)
import jax
import jax.numpy as jnp
from jax import lax
from jax.experimental import pallas as pl
from jax.experimental.pallas import tpu as pltpu
from jax.experimental.pallas import tpu_sc as plsc

_DIM = 32
_NC, _NS = 2, 16
_NW = _NC * _NS
_C = 512
_L = 200
_B = 4096
_VP = 1000064
_NBLK = _VP // 128
_BPW = (_NBLK + _NW - 1) // _NW


def _transpose_body(tblT_hbm, out_hbm, in_v, out_v, semi0, semi1, semo0,
                    semo1):
    wid = lax.axis_index("s") * _NC + lax.axis_index("c")
    lim = jnp.minimum(_NBLK, (wid + 1) * _BPW)
    semi = (semi0, semi1)
    semo = (semo0, semo1)

    iota16 = lax.iota(jnp.int32, 16)
    rowc = []
    colc = []
    for g in range(8):
        w = g * 16 + iota16
        rowc.append(lax.bitwise_and(w, 31))
        colc.append(lax.shift_right_logical(w, 5))

    def blk(bi):
        return wid * _BPW + bi

    def in_copy(bi, s):
        return pltpu.make_async_copy(
            tblT_hbm.at[:, pl.ds(blk(bi) * 128, 128)], in_v.at[s], semi[s])

    def out_copy(bi, s):
        return pltpu.make_async_copy(
            out_v.at[s], out_hbm.at[pl.ds(blk(bi) * 32, 32)], semo[s])

    def transpose(s):
        @pl.loop(0, _DIM)
        def _fat(f):
            colb = jnp.broadcast_to(4 * f, (16,))
            for g in range(8):
                vec = plsc.load_gather(in_v.at[s], [rowc[g], colb + colc[g]])
                out_v[s, f, pl.ds(g * 16, 16)] = vec

    @pl.when(blk(0) < lim)
    def _():
        in_copy(0, 0).start()

    @pl.when(blk(1) < lim)
    def _():
        in_copy(1, 1).start()

    @pl.loop(0, (_BPW + 2) // 2)
    def _pair(t):
        for b in range(2):
            s = b
            bi = 2 * t + b

            @pl.when(blk(bi) < lim)
            def _():
                in_copy(bi, s).wait()

                @pl.when(bi >= 2)
                def _():
                    out_copy(bi - 2, s).wait()

                transpose(s)

                @pl.when(blk(bi + 2) < lim)
                def _():
                    in_copy(bi + 2, s).start()

                out_copy(bi, s).start()

    n_valid = lim - wid * _BPW
    for s in range(2):
        last = ((n_valid - 1 - s) // 2) * 2 + s

        @pl.when(n_valid > s)
        def _():
            out_copy(last, s).wait()


def _gather_body(idx_hbm, table_hbm, out_hbm, idx_v, rows_v, out_t,
                 semi0, semi1, semg0, semg1, semo0, semo1):
    n_rows = idx_hbm.shape[0]
    r_per_w = n_rows // _NW
    n_chunks = r_per_w // _C
    wid = lax.axis_index("s") * _NC + lax.axis_index("c")
    base = wid * r_per_w

    semi = (semi0, semi1)
    semg = (semg0, semg1)
    semo = (semo0, semo1)

    iota16 = lax.iota(jnp.int32, 16)
    dconst = []
    for h in range(2):
        d = h * 16 + iota16
        dconst.append(lax.shift_right_logical(d, 3) * 4096 +
                      lax.bitwise_and(d, 7) * 128)

    def idx_copy(c, s):
        return pltpu.make_async_copy(
            idx_hbm.at[pl.ds(base + c * _C, _C)], idx_v.at[s], semi[s])

    def fire_gather(s):
        pltpu.async_copy(table_hbm.at[idx_v.at[s]], rows_v.at[s], semg[s])

    def drain_gather(s):
        pltpu.make_async_copy(table_hbm.at[pl.ds(0, _C)], rows_v.at[s],
                              semg[s]).wait()

    def out_pieces(c, s):
        flat0 = base + c * _C
        l = flat0 // _B
        j0 = (flat0 % _B) // 128
        return [pltpu.make_async_copy(
                    out_t.at[s, pl.ds(i * 4096, 4096)],
                    out_hbm.at[l, pl.ds(i * 32768 + j0 * 1024, 4096)],
                    semo[s])
                for i in range(4)]

    def out_start(c, s):
        for p in out_pieces(c, s):
            p.start()

    def out_wait(c, s):
        for p in out_pieces(c, s):
            p.wait()

    def transpose(s):
        @pl.loop(0, _C, unroll=2)
        def _row(r):
            rbase = jnp.broadcast_to(
                lax.shift_right_logical(r, 7) * 1024 +
                lax.bitwise_and(r, 127), (16,))
            for h in range(2):
                vec = rows_v[s, r, pl.ds(h * 16, 16)]
                plsc.store_scatter(out_t.at[s], [rbase + dconst[h]], vec)

    idx_copy(0, 0).start()
    idx_copy(1, 1).start()
    idx_copy(0, 0).wait()
    fire_gather(0)

    @pl.loop(0, n_chunks // 2)
    def _pair(t):
        for b in range(2):
            c = 2 * t + b
            s = b
            drain_gather(s)

            @pl.when(c + 2 < n_chunks)
            def _():
                idx_copy(c + 2, s).start()

            @pl.when(c + 1 < n_chunks)
            def _():
                idx_copy(c + 1, 1 - s).wait()
                fire_gather(1 - s)

            @pl.when(c >= 2)
            def _():
                out_wait(c - 2, s)

            transpose(s)
            out_start(c, s)

    out_wait(n_chunks - 2, 0)
    out_wait(n_chunks - 1, 1)


def kernel(inp, table):
    b, l = inp.shape
    n = b * l
    idx = inp.T.reshape(n).astype(jnp.int32)
    mesh = plsc.VectorSubcoreMesh(core_axis_name="c", subcore_axis_name="s")

    tbl_rows = pl.kernel(
        _transpose_body,
        out_type=jax.ShapeDtypeStruct((_VP // 4, 128), table.dtype),
        mesh=mesh,
        scratch_types=[
            pltpu.VMEM((2, _DIM, 128), jnp.float32),
            pltpu.VMEM((2, _DIM, 128), jnp.float32),
            pltpu.SemaphoreType.DMA,
            pltpu.SemaphoreType.DMA,
            pltpu.SemaphoreType.DMA,
            pltpu.SemaphoreType.DMA,
        ],
        compiler_params=pltpu.CompilerParams(use_tc_tiling_on_sc=True,
                                             needs_layout_passes=False),
    )(table.T)

    out2 = pl.kernel(
        _gather_body,
        out_type=jax.ShapeDtypeStruct((_L, 131072), table.dtype),
        mesh=mesh,
        scratch_types=[
            pltpu.VMEM((2, _C), jnp.int32),
            pltpu.VMEM((2, _C, _DIM), jnp.float32),
            pltpu.VMEM((2, 16384), jnp.float32),
            pltpu.SemaphoreType.DMA,
            pltpu.SemaphoreType.DMA,
            pltpu.SemaphoreType.DMA,
            pltpu.SemaphoreType.DMA,
            pltpu.SemaphoreType.DMA,
            pltpu.SemaphoreType.DMA,
        ],
        compiler_params=pltpu.CompilerParams(use_tc_tiling_on_sc=False,
                                             needs_layout_passes=False),
    )(idx, tbl_rows.reshape(_VP, _DIM))

    return (out2.reshape(_L, 4, 32, 8, 128)
                .transpose(2, 4, 0, 1, 3).reshape(b, l, _DIM))
